# baseline (speedup 1.0000x reference)
import functools

import jax
import jax.numpy as jnp
import numpy as np
from jax import lax
from jax.experimental import pallas as pl
from jax.experimental.pallas import tpu as pltpu
from jax.experimental.pallas import tpu_sc as plsc

_BATCH = 4096
_HIST = 200
_HP = 208
_NV = _HP // 16
_D = 64
_NC = 2
_NS = 16
_NW = _NC * _NS
_EPW = _BATCH // _NW
_IPW = _EPW * _HP
_OPW = _EPW * _D
_NBUF = 4

_CR = 504
_CW = 31248
_CT = _NW * _CW


def _unpack2(chunk):
    return plsc.unpack(chunk, format=plsc.PackFormat.INTERLEAVED)


def _pack2(a, b):
    return plsc.pack(a, b, format=plsc.PackFormat.INTERLEAVED)


def _sc_cast_body(tf_hbm, tb_hbm, in0, in1, ou0, ou1,
                  sem0, sem1, osem0, osem1):
    ins, ous = (in0, in1), (ou0, ou1)
    sems, osems = (sem0, sem1), (osem0, osem1)
    wid = lax.axis_index("s") * _NC + lax.axis_index("c")
    base = wid * _CW
    nblk = _CW // _CR

    def cvt_rows(src, dst, n):
        def crow(r, c):
            a0 = src[r, pl.ds(0, 16)]
            a1 = src[r, pl.ds(16, 16)]
            a2 = src[r, pl.ds(32, 16)]
            a3 = src[r, pl.ds(48, 16)]
            dst[r, pl.ds(0, 32)] = _pack2(a0, a1)
            dst[r, pl.ds(32, 32)] = _pack2(a2, a3)
            return c
        lax.fori_loop(0, n, crow, 0)

    for e in range(2):
        pltpu.async_copy(
            tf_hbm.at[pl.ds(base + e * _CR, _CR)], ins[e], sems[e])

    def body(i, carry):
        for e in range(2):
            blk = 2 * i + e
            pltpu.make_async_copy(
                tf_hbm.at[pl.ds(0, _CR)], ins[e], sems[e]).wait()

            @pl.when(blk >= 2)
            def _(e=e):
                pltpu.make_async_copy(
                    ous[e], tb_hbm.at[pl.ds(0, _CR)], osems[e]).wait()

            cvt_rows(ins[e], ous[e], _CR)
            pltpu.async_copy(
                ous[e], tb_hbm.at[pl.ds(base + blk * _CR, _CR)], osems[e])

            @pl.when(blk + 2 < nblk)
            def _(e=e, blk=blk):
                pltpu.async_copy(
                    tf_hbm.at[pl.ds(base + (blk + 2) * _CR, _CR)],
                    ins[e], sems[e])
        return carry

    lax.fori_loop(0, nblk // 2, body, 0)
    for e in range(2):
        pltpu.make_async_copy(
            ous[e], tb_hbm.at[pl.ds(0, _CR)], osems[e]).wait()

    @pl.when(wid == 0)
    def _():
        pltpu.sync_copy(tf_hbm.at[pl.ds(_CT, 64)], in0.at[pl.ds(0, 64)])
        cvt_rows(in0, ou0, 64)
        pltpu.sync_copy(ou0.at[pl.ds(0, 64)], tb_hbm.at[pl.ds(_CT, 64)])


_sc_cast = functools.partial(
    pl.kernel,
    out_type=jax.ShapeDtypeStruct((1000000, _D), jnp.bfloat16),
    mesh=plsc.VectorSubcoreMesh(core_axis_name="c", subcore_axis_name="s"),
    compiler_params=pltpu.CompilerParams(
        needs_layout_passes=False, use_tc_tiling_on_sc=False),
    scratch_types=(
        [pltpu.VMEM((_CR, _D), jnp.float32)] * 2
        + [pltpu.VMEM((_CR, _D), jnp.bfloat16)] * 2
        + [pltpu.SemaphoreType.DMA] * 4
    ),
)(_sc_cast_body)


def _sc_pool_body(xp_hbm, table_hbm, out_hbm,
                  idx_v, buf0, buf1, buf2, buf3, row0_v, out_v,
                  sem0, sem1, sem2, sem3):
    bufs = (buf0, buf1, buf2, buf3)
    sems = (sem0, sem1, sem2, sem3)
    wid = lax.axis_index("s") * _NC + lax.axis_index("c")

    pltpu.sync_copy(xp_hbm.at[pl.ds(wid * _IPW, _IPW)], idx_v)
    pltpu.sync_copy(table_hbm.at[pl.ds(0, 8)], row0_v)

    r0a, r0b = _unpack2(row0_v[0, pl.ds(0, 32)])
    r0c, r0d = _unpack2(row0_v[0, pl.ds(32, 32)])
    row0 = (r0a, r0b, r0c, r0d)

    def fire(b, j):
        for k in range(_NV):
            ivec = idx_v[pl.ds(b * _HP + 16 * k, 16)]
            pltpu.async_copy(
                table_hbm.at[ivec], bufs[j].at[pl.ds(16 * k, 16)], sems[j])

    for j in range(_NBUF):
        fire(j, j)

    inv_n = jnp.float32(1.0 / _HIST)

    def elem(i, e):
        b = _NBUF * i + e
        buf = bufs[e]

        cntv = jnp.zeros((16,), jnp.int32)
        for k in range(_NV):
            c = idx_v[pl.ds(b * _HP + 16 * k, 16)]
            cntv += plsc.all_reduce_population_count(c == 0)

        pltpu.make_async_copy(
            table_hbm.at[pl.ds(0, _HP)], buf, sems[e]).wait()

        def row_add(jr, a, unroll=4):
            base = jr * unroll
            for u in range(unroll):
                lo = _unpack2(buf[base + u, pl.ds(0, 32)])
                hi = _unpack2(buf[base + u, pl.ds(32, 32)])
                a = (a[0] + lo[0], a[1] + lo[1],
                     a[2] + hi[0], a[3] + hi[1])
            return a

        acc = (jnp.zeros((16,), jnp.float32),) * 4
        acc = lax.fori_loop(0, _HP // 4, row_add, acc)

        cnt = cntv.astype(jnp.float32)
        for k in range(4):
            val = (acc[k] - cnt * row0[k]) * inv_n
            out_v[pl.ds(b * _D + 16 * k, 16)] = val

        @pl.when(b + _NBUF < _EPW)
        def _(e=e):
            fire(b + _NBUF, e)

    def body(i, carry):
        for e in range(_NBUF):
            elem(i, e)
        return carry

    lax.fori_loop(0, _EPW // _NBUF, body, 0)

    pltpu.sync_copy(out_v, out_hbm.at[pl.ds(wid * _OPW, _OPW)])


_sc_pool = functools.partial(
    pl.kernel,
    out_type=jax.ShapeDtypeStruct((_BATCH * _D,), jnp.float32),
    mesh=plsc.VectorSubcoreMesh(core_axis_name="c", subcore_axis_name="s"),
    compiler_params=pltpu.CompilerParams(
        needs_layout_passes=False, use_tc_tiling_on_sc=False),
    scratch_types=(
        [pltpu.VMEM((_IPW,), jnp.int32)]
        + [pltpu.VMEM((_HP, _D), jnp.bfloat16)] * _NBUF
        + [pltpu.VMEM((8, _D), jnp.bfloat16),
           pltpu.VMEM((_OPW,), jnp.float32)]
        + [pltpu.SemaphoreType.DMA] * _NBUF
    ),
)(_sc_pool_body)


def _mlp_body(x_ref, w1_ref, b1_ref, w2_ref, b2_ref, o_ref):
    h = jnp.dot(x_ref[...], w1_ref[...], preferred_element_type=jnp.float32)
    h = jnp.maximum(h + b1_ref[...], 0.0)
    o_ref[...] = (
        jnp.dot(h, w2_ref[...], preferred_element_type=jnp.float32)
        + b2_ref[...])


def kernel(x, table, W1, b1, W2, b2):
    xi = x.astype(jnp.int32)
    xp = jnp.pad(xi, ((0, 0), (0, _HP - _HIST))).reshape(-1)
    tb = _sc_cast(table)

    pooled = _sc_pool(xp, tb).reshape(_BATCH, _D)

    ncls = W2.shape[1]
    w1p = W1
    w2p = jnp.pad(W2, ((0, 0), (0, _D - ncls)))
    b2p = jnp.pad(b2, (0, _D - ncls)).reshape(1, _D)
    out = pl.pallas_call(
        _mlp_body,
        out_shape=jax.ShapeDtypeStruct((_BATCH, _D), jnp.float32),
    )(pooled, w1p, b1.reshape(1, -1), w2p, b2p)
    return out[:, :ncls]
